# baseline (speedup 1.0000x reference)
import functools

import jax
import jax.numpy as jnp
from jax import lax
from jax.experimental import pallas as pl
from jax.experimental.pallas import tpu as pltpu
from jax.experimental.pallas import tpu_sc as plsc

N = 10000
E = 160000
DIN = 256
H = 512
DE = 16

NTILES = 32
P = 320
NP = NTILES * P
C = 2000
NCHUNK = E // C
NG = C // 16
CAP = 8192
G = 64
NBUF = 4
NC4 = H // 128

BN = 2048
GRID_N = NP // BN


def _proj_body(apply_relu, x_ref, w_ref, asd_ref, b_ref, h_ref, sd_ref):
    xin = x_ref[...]
    if apply_relu:
        xin = jnp.maximum(xin + b_ref[...], 0.0)
    h = jnp.dot(xin, w_ref[...], preferred_element_type=jnp.float32)
    h_ref[...] = h
    sd_ref[...] = jnp.dot(h, asd_ref[...], preferred_element_type=jnp.float32)


def _projection(xin, W, a_s, a_d, b, apply_relu):
    K = xin.shape[1]
    asd = jnp.zeros((H, 128), jnp.float32).at[:, 0].set(a_s).at[:, 1].set(a_d)
    h, sd = pl.pallas_call(
        functools.partial(_proj_body, apply_relu),
        grid=(GRID_N,),
        in_specs=[
            pl.BlockSpec((BN, K), lambda i: (i, 0)),
            pl.BlockSpec((K, H), lambda i: (0, 0)),
            pl.BlockSpec((H, 128), lambda i: (0, 0)),
            pl.BlockSpec((1, H), lambda i: (0, 0)),
        ],
        out_specs=(
            pl.BlockSpec((BN, H), lambda i: (i, 0)),
            pl.BlockSpec((BN, 128), lambda i: (i, 0)),
        ),
        out_shape=(
            jax.ShapeDtypeStruct((NP, H), jnp.float32),
            jax.ShapeDtypeStruct((NP, 128), jnp.float32),
        ),
    )(xin, W, asd, b.reshape(1, H))
    return h, sd[:, 0], sd[:, 1]


EP = 163840
EB = 8192
GRID_E = EP // EB


ER = EP // 64


def _el_body(ea_ref, we1_ref, ae1_ref, we2_ref, ae2_ref, o_ref):
    wv1 = jnp.sum(we1_ref[...] * ae1_ref[...], axis=1)
    wv2 = jnp.sum(we2_ref[...] * ae2_ref[...], axis=1)
    jj = lax.broadcasted_iota(jnp.int32, (64 * DE, 128), 0)
    mm = lax.broadcasted_iota(jnp.int32, (64 * DE, 128), 1)
    w1b = jnp.broadcast_to(wv1.reshape(1, DE, 1),
                           (64, DE, 128)).reshape(64 * DE, 128)
    w2b = jnp.broadcast_to(wv2.reshape(1, DE, 1),
                           (64, DE, 128)).reshape(64 * DE, 128)
    blk = jj >> 4
    wv = jnp.where(blk == mm, w1b, 0.0) + jnp.where(blk == mm - 64, w2b, 0.0)
    o_ref[...] = jnp.dot(ea_ref[...], wv, preferred_element_type=jnp.float32)


def _edge_logits(ea_r, We1, ae1, We2, ae2):
    o = pl.pallas_call(
        _el_body,
        grid=(1,),
        in_specs=[
            pl.BlockSpec((ER, 64 * DE), lambda i: (0, 0)),
            pl.BlockSpec((DE, H), lambda i: (0, 0)),
            pl.BlockSpec((1, H), lambda i: (0, 0)),
            pl.BlockSpec((DE, H), lambda i: (0, 0)),
            pl.BlockSpec((1, H), lambda i: (0, 0)),
        ],
        out_specs=pl.BlockSpec((ER, 128), lambda i: (0, 0)),
        out_shape=jax.ShapeDtypeStruct((ER, 128), jnp.float32),
    )(ea_r, We1, ae1.reshape(1, H), We2, ae2.reshape(1, H))
    return o[:, :64].reshape(EP), o[:, 64:].reshape(EP)


def _sc_mesh():
    return plsc.VectorSubcoreMesh(core_axis_name="c", subcore_axis_name="s",
                                  num_cores=2, num_subcores=16)


_SC_PARAMS = pltpu.CompilerParams(needs_layout_passes=False)


def _alpha_pass(cnt, lanes, exc_v, pkc_v, den_v):
    def alpha_grp(i, carry):
        o = i * 16
        m = (o + lanes) < cnt
        ev = exc_v[pl.ds(o, 16)]
        dof = pkc_v[pl.ds(o, 16)] & 511
        dg = plsc.load_gather(den_v, [dof])
        al = ev / (dg + 1e-16)
        exc_v[pl.ds(o, 16)] = jnp.where(m, al, 0.0)
        return carry

    plsc.parallel_loop(0, CAP // 16, unroll=4)(
        lambda i: alpha_grp(i, 0) and None)


def _aggregate(cnt, lanes, lo, h4_hbm, oc_hbm, exc_v, pkc_v,
               idxs, rowss, acc_v, sems):
    zf = jnp.zeros((16,), jnp.float32)
    nch = (cnt + G - 1) // G

    def chunk_c(c, carry_c):
        def zacc(i, carry):
            for q in range(8):
                acc_v[i, pl.ds(q * 16, 16)] = zf
            return carry

        lax.fori_loop(0, P, zacc, 0, unroll=4)

        def build_idx(kk, idx_ref):
            gb = kk * G
            for j in range(G // 16):
                o = gb + j * 16
                m = (o + lanes) < cnt
                src4 = (pkc_v[pl.ds(o, 16)] >> 9) * 4
                iv = jnp.where(m, src4 + c, 0)
                idx_ref[pl.ds(j * 16, 16)] = iv

        def g_start(idx_ref, rows_ref, sem):
            pltpu.async_copy(h4_hbm.at[idx_ref], rows_ref, sem)

        def g_wait(idx_ref, rows_ref, sem):
            pltpu.make_async_copy(h4_hbm.at[idx_ref], rows_ref, sem).wait()

        def process(kk, rows_ref):
            gb = kk * G

            @plsc.parallel_loop(0, G // 16, unroll=4)
            def egrp(g2):
                ob = gb + g2 * 16
                av = exc_v[pl.ds(ob, 16)]
                basev = pkc_v[pl.ds(ob, 16)] & 511
                for j in range(16):
                    a = av[j]
                    base = basev[j]
                    row = g2 * 16 + j
                    for q in range(8):
                        r = rows_ref[row, pl.ds(q * 16, 16)]
                        plsc.addupdate(acc_v.at[base, pl.ds(q * 16, 16)],
                                       r * a)

        for b in range(NBUF):
            @pl.when(b < nch)
            def _(b=b):
                build_idx(b, idxs[b])
                g_start(idxs[b], rowss[b], sems[b])

        def p2_outer(j, carry):
            for b in range(NBUF):
                ch = NBUF * j + b

                @pl.when(ch < nch)
                def _(b=b, ch=ch):
                    g_wait(idxs[b], rowss[b], sems[b])
                    process(ch, rowss[b])

                    @pl.when(ch + NBUF < nch)
                    def _():
                        build_idx(ch + NBUF, idxs[b])
                        g_start(idxs[b], rowss[b], sems[b])

            return carry

        lax.fori_loop(0, (nch + NBUF - 1) // NBUF, p2_outer, 0)
        pltpu.sync_copy(acc_v,
                        oc_hbm.at[pl.ds(lo, P), pl.ds(c * 128, 128)])
        return carry_c

    lax.fori_loop(0, NC4, chunk_c, 0)


def _edge_kernel1(h4, s, d, el, el2, src, dst):
    @functools.partial(
        pl.kernel,
        mesh=_sc_mesh(),
        compiler_params=_SC_PARAMS,
        out_type=(
            jax.ShapeDtypeStruct((NP, H), jnp.float32),
            jax.ShapeDtypeStruct((NTILES * CAP,), jnp.int32),
            jax.ShapeDtypeStruct((NTILES * CAP,), jnp.float32),
            jax.ShapeDtypeStruct((NTILES * 16,), jnp.int32),
        ),
        scratch_types=[
            pltpu.VMEM((NP,), jnp.float32),
            pltpu.VMEM((P,), jnp.float32),
            pltpu.VMEM((C,), jnp.int32),
            pltpu.VMEM((C,), jnp.int32),
            pltpu.VMEM((C,), jnp.float32),
            pltpu.VMEM((C,), jnp.float32),
            pltpu.VMEM((C,), jnp.int32),
            pltpu.VMEM((C,), jnp.int32),
            pltpu.VMEM((C,), jnp.float32),
            pltpu.VMEM((C,), jnp.float32),
            pltpu.VMEM((P,), jnp.float32),
            pltpu.VMEM((CAP,), jnp.float32),
            pltpu.VMEM((CAP,), jnp.int32),
            pltpu.VMEM((CAP,), jnp.float32),
            pltpu.VMEM((G,), jnp.int32),
            pltpu.VMEM((G,), jnp.int32),
            pltpu.VMEM((G,), jnp.int32),
            pltpu.VMEM((G,), jnp.int32),
            pltpu.VMEM((G, 128), jnp.float32),
            pltpu.VMEM((G, 128), jnp.float32),
            pltpu.VMEM((G, 128), jnp.float32),
            pltpu.VMEM((G, 128), jnp.float32),
            pltpu.VMEM((P, 128), jnp.float32),
            pltpu.SemaphoreType.DMA,
            pltpu.SemaphoreType.DMA,
            pltpu.SemaphoreType.DMA,
            pltpu.SemaphoreType.DMA,
            pltpu.SemaphoreType.DMA,
            pltpu.SemaphoreType.DMA,
        ],
    )
    def k(h4_hbm, s_hbm, d_hbm, el_hbm, el2_hbm, src_hbm, dst_hbm,
          oc_hbm, pk_hbm, el2c_hbm, cnt_hbm,
          s_v, d_v, src_va, dst_va, el_va, el2_va,
          src_vb, dst_vb, el_vb, el2_vb,
          den_v, exc_v, pkc_v, el2c_v,
          idx_v0, idx_v1, idx_v2, idx_v3,
          rows_v0, rows_v1, rows_v2, rows_v3,
          acc_v, sema, semb, semc, semd, seme, semf):
        wid = lax.axis_index("s") * 2 + lax.axis_index("c")
        lo = wid * P
        pltpu.async_copy(s_hbm, s_v, sema)
        pltpu.async_copy(d_hbm.at[pl.ds(lo, P)], d_v, sema)
        zf = jnp.zeros((16,), jnp.float32)
        zi = jnp.zeros((16,), jnp.int32)
        lanes = lax.iota(jnp.int32, 16)

        def zden(i, carry):
            den_v[pl.ds(i * 16, 16)] = zf
            return carry

        lax.fori_loop(0, P // 16, zden, 0)

        def zcap(i, carry):
            exc_v[pl.ds(i * 16, 16)] = zf
            pkc_v[pl.ds(i * 16, 16)] = zi
            return carry

        lax.fori_loop(0, CAP // 16, zcap, 0, unroll=8)
        pltpu.make_async_copy(s_hbm, s_v, sema).wait()
        pltpu.make_async_copy(d_hbm.at[pl.ds(lo, P)], d_v, sema).wait()

        def p1_start(ch, bufs, sem):
            base = ch * C
            pltpu.async_copy(src_hbm.at[pl.ds(base, C)], bufs[0], sem)
            pltpu.async_copy(dst_hbm.at[pl.ds(base, C)], bufs[1], sem)
            pltpu.async_copy(el_hbm.at[pl.ds(base, C)], bufs[2], sem)
            pltpu.async_copy(el2_hbm.at[pl.ds(base, C)], bufs[3], sem)

        def p1_wait(ch, bufs, sem):
            base = ch * C
            pltpu.make_async_copy(src_hbm.at[pl.ds(base, C)], bufs[0],
                                  sem).wait()
            pltpu.make_async_copy(dst_hbm.at[pl.ds(base, C)], bufs[1],
                                  sem).wait()
            pltpu.make_async_copy(el_hbm.at[pl.ds(base, C)], bufs[2],
                                  sem).wait()
            pltpu.make_async_copy(el2_hbm.at[pl.ds(base, C)], bufs[3],
                                  sem).wait()

        def p1_process(bufs, cnt):
            src_c, dst_c, el_c, el2_c = bufs

            def grp(g, cnt):
                o = g * 16
                sv = src_c[pl.ds(o, 16)]
                dv = dst_c[pl.ds(o, 16)]
                ev = el_c[pl.ds(o, 16)]
                e2v = el2_c[pl.ds(o, 16)]
                own = (dv >= lo) & (dv < lo + P)
                dof = jnp.where(own, dv - lo, 0)
                sg = plsc.load_gather(s_v, [sv])
                dg = plsc.load_gather(d_v, [dof])
                lg = sg + dg + ev
                lg = jnp.maximum(lg, lg * 0.2)
                ex = jnp.exp(lg)
                plsc.addupdate_scatter(den_v, [dof], ex, mask=own)
                plsc.store_compressed(exc_v.at[pl.ds(cnt, 16)], ex, mask=own)
                plsc.store_compressed(pkc_v.at[pl.ds(cnt, 16)],
                                      sv * 512 + dof, mask=own)
                plsc.store_compressed(el2c_v.at[pl.ds(cnt, 16)], e2v,
                                      mask=own)
                return cnt + plsc.all_reduce_population_count(own)[0]

            return plsc.parallel_loop(0, NG, unroll=4, carry=cnt)(grp)

        bufs_a = (src_va, dst_va, el_va, el2_va)
        bufs_b = (src_vb, dst_vb, el_vb, el2_vb)
        p1_start(0, bufs_a, sema)

        def p1_outer(j, cnt):
            c0 = 2 * j
            p1_wait(c0, bufs_a, sema)
            p1_start(c0 + 1, bufs_b, semb)
            cnt = p1_process(bufs_a, cnt)

            @pl.when(j < NCHUNK // 2 - 1)
            def _():
                p1_start(c0 + 2, bufs_a, sema)

            p1_wait(c0 + 1, bufs_b, semb)
            return p1_process(bufs_b, cnt)

        cnt = lax.fori_loop(0, NCHUNK // 2, p1_outer, jnp.int32(0))

        pltpu.async_copy(pkc_v, pk_hbm.at[pl.ds(wid * CAP, CAP)], semb)
        pltpu.async_copy(el2c_v, el2c_hbm.at[pl.ds(wid * CAP, CAP)], semb)
        idx_v0[pl.ds(0, 16)] = jnp.full((16,), cnt, jnp.int32)
        pltpu.async_copy(idx_v0.at[pl.ds(0, 16)],
                         cnt_hbm.at[pl.ds(wid * 16, 16)], semb)

        _alpha_pass(cnt, lanes, exc_v, pkc_v, den_v)

        pltpu.make_async_copy(pkc_v, pk_hbm.at[pl.ds(wid * CAP, CAP)],
                              semb).wait()
        pltpu.make_async_copy(el2c_v, el2c_hbm.at[pl.ds(wid * CAP, CAP)],
                              semb).wait()
        pltpu.make_async_copy(idx_v0.at[pl.ds(0, 16)],
                              cnt_hbm.at[pl.ds(wid * 16, 16)], semb).wait()

        _aggregate(cnt, lanes, lo, h4_hbm, oc_hbm, exc_v, pkc_v,
                   (idx_v0, idx_v1, idx_v2, idx_v3),
                   (rows_v0, rows_v1, rows_v2, rows_v3),
                   acc_v, (semc, semd, seme, semf))

    return k(h4, s, d, el, el2, src, dst)


def _edge_kernel2(h4, s, d, pk, el2c, cntv):
    @functools.partial(
        pl.kernel,
        mesh=_sc_mesh(),
        compiler_params=_SC_PARAMS,
        out_type=jax.ShapeDtypeStruct((NP, H), jnp.float32),
        scratch_types=[
            pltpu.VMEM((NP,), jnp.float32),
            pltpu.VMEM((P,), jnp.float32),
            pltpu.VMEM((P,), jnp.float32),
            pltpu.VMEM((CAP,), jnp.float32),
            pltpu.VMEM((CAP,), jnp.int32),
            pltpu.VMEM((CAP,), jnp.float32),
            pltpu.VMEM((G,), jnp.int32),
            pltpu.VMEM((G,), jnp.int32),
            pltpu.VMEM((G,), jnp.int32),
            pltpu.VMEM((G,), jnp.int32),
            pltpu.VMEM((G, 128), jnp.float32),
            pltpu.VMEM((G, 128), jnp.float32),
            pltpu.VMEM((G, 128), jnp.float32),
            pltpu.VMEM((G, 128), jnp.float32),
            pltpu.VMEM((P, 128), jnp.float32),
            pltpu.SemaphoreType.DMA,
            pltpu.SemaphoreType.DMA,
            pltpu.SemaphoreType.DMA,
            pltpu.SemaphoreType.DMA,
            pltpu.SemaphoreType.DMA,
            pltpu.SemaphoreType.DMA,
        ],
    )
    def k(h4_hbm, s_hbm, d_hbm, pk_hbm, el2c_hbm, cnt_hbm, oc_hbm,
          s_v, d_v, den_v, exc_v, pkc_v, el2c_v,
          idx_v0, idx_v1, idx_v2, idx_v3,
          rows_v0, rows_v1, rows_v2, rows_v3,
          acc_v, sema, semb, semc, semd, seme, semf):
        wid = lax.axis_index("s") * 2 + lax.axis_index("c")
        lo = wid * P
        pltpu.async_copy(s_hbm, s_v, sema)
        pltpu.async_copy(d_hbm.at[pl.ds(lo, P)], d_v, sema)
        pltpu.async_copy(pk_hbm.at[pl.ds(wid * CAP, CAP)], pkc_v, semb)
        pltpu.async_copy(el2c_hbm.at[pl.ds(wid * CAP, CAP)], el2c_v, semb)
        pltpu.async_copy(cnt_hbm.at[pl.ds(wid * 16, 16)],
                         idx_v0.at[pl.ds(0, 16)], semb)
        zf = jnp.zeros((16,), jnp.float32)
        lanes = lax.iota(jnp.int32, 16)

        def zden(i, carry):
            den_v[pl.ds(i * 16, 16)] = zf
            return carry

        lax.fori_loop(0, P // 16, zden, 0)
        pltpu.make_async_copy(s_hbm, s_v, sema).wait()
        pltpu.make_async_copy(d_hbm.at[pl.ds(lo, P)], d_v, sema).wait()
        pltpu.make_async_copy(pk_hbm.at[pl.ds(wid * CAP, CAP)], pkc_v,
                              semb).wait()
        pltpu.make_async_copy(el2c_hbm.at[pl.ds(wid * CAP, CAP)], el2c_v,
                              semb).wait()
        pltpu.make_async_copy(cnt_hbm.at[pl.ds(wid * 16, 16)],
                              idx_v0.at[pl.ds(0, 16)], semb).wait()
        cnt = idx_v0[pl.ds(0, 16)][0]


        def grp(g, carry):
            o = g * 16
            m = (o + lanes) < cnt
            pkv = pkc_v[pl.ds(o, 16)]
            dof = pkv & 511
            srcv = pkv >> 9
            e2v = el2c_v[pl.ds(o, 16)]
            sg = plsc.load_gather(s_v, [srcv])
            dg = plsc.load_gather(d_v, [dof])
            lg = sg + dg + e2v
            lg = jnp.maximum(lg, lg * 0.2)
            ex = jnp.where(m, jnp.exp(lg), 0.0)
            plsc.addupdate_scatter(den_v, [dof], ex, mask=m)
            exc_v[pl.ds(o, 16)] = ex
            return carry

        plsc.parallel_loop(0, CAP // 16, unroll=4)(
            lambda g: grp(g, 0) and None)

        _alpha_pass(cnt, lanes, exc_v, pkc_v, den_v)
        _aggregate(cnt, lanes, lo, h4_hbm, oc_hbm, exc_v, pkc_v,
                   (idx_v0, idx_v1, idx_v2, idx_v3),
                   (rows_v0, rows_v1, rows_v2, rows_v3),
                   acc_v, (semc, semd, seme, semf))

    return k(h4, s, d, pk, el2c, cntv)


def kernel(x, edge_index, edge_attr, W1, a_src1, a_dst1, We1, ae1, b1,
           W2, a_src2, a_dst2, We2, ae2, b2):
    x_pad = jnp.zeros((NP, DIN), jnp.float32).at[:N].set(x)
    src = edge_index[0]
    dst = edge_index[1]
    ea_pad = jnp.zeros((EP, DE), jnp.float32).at[:E].set(edge_attr)
    el1, el2 = _edge_logits(ea_pad.reshape(ER, 64 * DE), We1, ae1, We2, ae2)

    h1, s1, d1 = _projection(x_pad, W1, a_src1, a_dst1,
                             jnp.zeros((H,), jnp.float32), False)
    o1, pk, el2c, cntv = _edge_kernel1(h1.reshape(NP * NC4, 128), s1, d1,
                                       el1, el2, src, dst)

    h2, s2, d2 = _projection(o1, W2, a_src2, a_dst2, b1, True)
    oc2 = _edge_kernel2(h2.reshape(NP * NC4, 128), s2, d2, pk, el2c, cntv)
    out = oc2[:N] + b2
    return out


# speedup vs baseline: 1.2610x; 1.2610x over previous
import functools

import jax
import jax.numpy as jnp
from jax import lax
from jax.experimental import pallas as pl
from jax.experimental.pallas import tpu as pltpu
from jax.experimental.pallas import tpu_sc as plsc

N = 10000
E = 160000
DIN = 256
H = 512
DE = 16

NTILES = 32
P = 320
NP = NTILES * P
C = 2000
NCHUNK = E // C
NG = C // 16
CAP = 8192
G = 64
NBUF = 4
NC4 = H // 128

BN = 2048
GRID_N = NP // BN


def _proj_body(apply_relu, x_ref, w_ref, asd_ref, b_ref, h_ref, sd_ref):
    xin = x_ref[...]
    if apply_relu:
        xin = jnp.maximum(xin + b_ref[...], 0.0)
    h = jnp.dot(xin, w_ref[...], preferred_element_type=jnp.float32)
    h_ref[...] = h
    sd_ref[...] = jnp.dot(h, asd_ref[...], preferred_element_type=jnp.float32)


def _projection(xin, W, a_s, a_d, b, apply_relu):
    K = xin.shape[1]
    asd = jnp.zeros((H, 128), jnp.float32).at[:, 0].set(a_s).at[:, 1].set(a_d)
    h, sd = pl.pallas_call(
        functools.partial(_proj_body, apply_relu),
        grid=(GRID_N,),
        in_specs=[
            pl.BlockSpec((BN, K), lambda i: (i, 0)),
            pl.BlockSpec((K, H), lambda i: (0, 0)),
            pl.BlockSpec((H, 128), lambda i: (0, 0)),
            pl.BlockSpec((1, H), lambda i: (0, 0)),
        ],
        out_specs=(
            pl.BlockSpec((BN, H), lambda i: (i, 0)),
            pl.BlockSpec((BN, 128), lambda i: (i, 0)),
        ),
        out_shape=(
            jax.ShapeDtypeStruct((NP, H), jnp.float32),
            jax.ShapeDtypeStruct((NP, 128), jnp.float32),
        ),
    )(xin, W, asd, b.reshape(1, H))
    return h, sd[:, 0], sd[:, 1]


EP = 163840
EB = 8192
GRID_E = EP // EB


ER = EP // 64


def _el_body(ea_ref, we1_ref, ae1_ref, we2_ref, ae2_ref, o_ref):
    wv1 = jnp.sum(we1_ref[...] * ae1_ref[...], axis=1)
    wv2 = jnp.sum(we2_ref[...] * ae2_ref[...], axis=1)
    jj = lax.broadcasted_iota(jnp.int32, (64 * DE, 128), 0)
    mm = lax.broadcasted_iota(jnp.int32, (64 * DE, 128), 1)
    w1b = jnp.broadcast_to(wv1.reshape(1, DE, 1),
                           (64, DE, 128)).reshape(64 * DE, 128)
    w2b = jnp.broadcast_to(wv2.reshape(1, DE, 1),
                           (64, DE, 128)).reshape(64 * DE, 128)
    blk = jj >> 4
    wv = jnp.where(blk == mm, w1b, 0.0) + jnp.where(blk == mm - 64, w2b, 0.0)
    o_ref[...] = jnp.dot(ea_ref[...], wv, preferred_element_type=jnp.float32)


def _edge_logits(ea_r, We1, ae1, We2, ae2):
    o = pl.pallas_call(
        _el_body,
        grid=(1,),
        in_specs=[
            pl.BlockSpec((ER, 64 * DE), lambda i: (0, 0)),
            pl.BlockSpec((DE, H), lambda i: (0, 0)),
            pl.BlockSpec((1, H), lambda i: (0, 0)),
            pl.BlockSpec((DE, H), lambda i: (0, 0)),
            pl.BlockSpec((1, H), lambda i: (0, 0)),
        ],
        out_specs=pl.BlockSpec((ER, 128), lambda i: (0, 0)),
        out_shape=jax.ShapeDtypeStruct((ER, 128), jnp.float32),
    )(ea_r, We1, ae1.reshape(1, H), We2, ae2.reshape(1, H))
    return o[:, :64].reshape(EP), o[:, 64:].reshape(EP)


def _sc_mesh():
    return plsc.VectorSubcoreMesh(core_axis_name="c", subcore_axis_name="s",
                                  num_cores=2, num_subcores=16)


_SC_PARAMS = pltpu.CompilerParams(needs_layout_passes=False)


def _alpha_pass(cnt, lanes, exc_v, pkc_v, den_v):
    def alpha_grp(i, carry):
        o = i * 16
        m = (o + lanes) < cnt
        ev = exc_v[pl.ds(o, 16)]
        dof = pkc_v[pl.ds(o, 16)] & 511
        dg = plsc.load_gather(den_v, [dof])
        al = ev / (dg + 1e-16)
        exc_v[pl.ds(o, 16)] = jnp.where(m, al, 0.0)
        return carry

    plsc.parallel_loop(0, CAP // 16, unroll=4)(
        lambda i: alpha_grp(i, 0) and None)


def _aggregate(cnt, lanes, lo, h4_hbm, oc_hbm, exc_v, pkc_v,
               idxs, rowss, acc_v, sems):
    zf = jnp.zeros((16,), jnp.float32)
    nch = (cnt + G - 1) // G

    def chunk_c(c, carry_c):
        def zacc(i, carry):
            acc_v[pl.ds(i * 16, 16)] = zf
            return carry

        lax.fori_loop(0, P * 8, zacc, 0, unroll=8)

        def build_idx(kk, idx_ref):
            gb = kk * G
            for j in range(G // 16):
                o = gb + j * 16
                m = (o + lanes) < cnt
                src4 = (pkc_v[pl.ds(o, 16)] >> 9) * 4
                iv = jnp.where(m, src4 + c, 0)
                idx_ref[pl.ds(j * 16, 16)] = iv

        def g_start(idx_ref, rows_ref, sem):
            pltpu.async_copy(h4_hbm.at[idx_ref], rows_ref, sem)

        def g_wait(idx_ref, rows_ref, sem):
            pltpu.make_async_copy(h4_hbm.at[idx_ref], rows_ref, sem).wait()

        def process(kk, rows_ref):
            gb = kk * G

            @plsc.parallel_loop(0, G // 16, unroll=2)
            def egrp(g2):
                ob = gb + g2 * 16
                av = exc_v[pl.ds(ob, 16)]
                basev = (pkc_v[pl.ds(ob, 16)] & 511) * 128
                for j in range(16):
                    a = av[j]
                    base = basev[j]
                    row = g2 * 16 + j
                    for q in range(8):
                        r = rows_ref[row, pl.ds(q * 16, 16)]
                        plsc.addupdate(acc_v.at[pl.ds(base + q * 16, 16)],
                                       r * a)

        for b in range(NBUF):
            @pl.when(b < nch)
            def _(b=b):
                build_idx(b, idxs[b])
                g_start(idxs[b], rowss[b], sems[b])

        def p2_outer(j, carry):
            for b in range(NBUF):
                ch = NBUF * j + b

                @pl.when(ch < nch)
                def _(b=b, ch=ch):
                    g_wait(idxs[b], rowss[b], sems[b])
                    process(ch, rowss[b])

                    @pl.when(ch + NBUF < nch)
                    def _():
                        build_idx(ch + NBUF, idxs[b])
                        g_start(idxs[b], rowss[b], sems[b])

            return carry

        lax.fori_loop(0, (nch + NBUF - 1) // NBUF, p2_outer, 0)
        pltpu.sync_copy(acc_v,
                        oc_hbm.at[pl.ds((c * NP + lo) * 128, P * 128)])
        return carry_c

    lax.fori_loop(0, NC4, chunk_c, 0)


def _edge_kernel1(h4, s, d, el, el2, src, dst):
    @functools.partial(
        pl.kernel,
        mesh=_sc_mesh(),
        compiler_params=_SC_PARAMS,
        out_type=(
            jax.ShapeDtypeStruct((NC4 * NP * 128,), jnp.float32),
            jax.ShapeDtypeStruct((NTILES * CAP,), jnp.int32),
            jax.ShapeDtypeStruct((NTILES * CAP,), jnp.float32),
            jax.ShapeDtypeStruct((NTILES * 16,), jnp.int32),
        ),
        scratch_types=[
            pltpu.VMEM((NP,), jnp.float32),
            pltpu.VMEM((P,), jnp.float32),
            pltpu.VMEM((C,), jnp.int32),
            pltpu.VMEM((C,), jnp.int32),
            pltpu.VMEM((C,), jnp.float32),
            pltpu.VMEM((C,), jnp.float32),
            pltpu.VMEM((C,), jnp.int32),
            pltpu.VMEM((C,), jnp.int32),
            pltpu.VMEM((C,), jnp.float32),
            pltpu.VMEM((C,), jnp.float32),
            pltpu.VMEM((P,), jnp.float32),
            pltpu.VMEM((CAP,), jnp.float32),
            pltpu.VMEM((CAP,), jnp.int32),
            pltpu.VMEM((CAP,), jnp.float32),
            pltpu.VMEM((G,), jnp.int32),
            pltpu.VMEM((G,), jnp.int32),
            pltpu.VMEM((G,), jnp.int32),
            pltpu.VMEM((G,), jnp.int32),
            pltpu.VMEM((G, 128), jnp.float32),
            pltpu.VMEM((G, 128), jnp.float32),
            pltpu.VMEM((G, 128), jnp.float32),
            pltpu.VMEM((G, 128), jnp.float32),
            pltpu.VMEM((P * 128,), jnp.float32),
            pltpu.SemaphoreType.DMA,
            pltpu.SemaphoreType.DMA,
            pltpu.SemaphoreType.DMA,
            pltpu.SemaphoreType.DMA,
            pltpu.SemaphoreType.DMA,
            pltpu.SemaphoreType.DMA,
        ],
    )
    def k(h4_hbm, s_hbm, d_hbm, el_hbm, el2_hbm, src_hbm, dst_hbm,
          oc_hbm, pk_hbm, el2c_hbm, cnt_hbm,
          s_v, d_v, src_va, dst_va, el_va, el2_va,
          src_vb, dst_vb, el_vb, el2_vb,
          den_v, exc_v, pkc_v, el2c_v,
          idx_v0, idx_v1, idx_v2, idx_v3,
          rows_v0, rows_v1, rows_v2, rows_v3,
          acc_v, sema, semb, semc, semd, seme, semf):
        wid = lax.axis_index("s") * 2 + lax.axis_index("c")
        lo = wid * P
        pltpu.async_copy(s_hbm, s_v, sema)
        pltpu.async_copy(d_hbm.at[pl.ds(lo, P)], d_v, sema)
        zf = jnp.zeros((16,), jnp.float32)
        zi = jnp.zeros((16,), jnp.int32)
        lanes = lax.iota(jnp.int32, 16)

        def zden(i, carry):
            den_v[pl.ds(i * 16, 16)] = zf
            return carry

        lax.fori_loop(0, P // 16, zden, 0)

        def zcap(i, carry):
            exc_v[pl.ds(i * 16, 16)] = zf
            pkc_v[pl.ds(i * 16, 16)] = zi
            return carry

        lax.fori_loop(0, CAP // 16, zcap, 0, unroll=8)
        pltpu.make_async_copy(s_hbm, s_v, sema).wait()
        pltpu.make_async_copy(d_hbm.at[pl.ds(lo, P)], d_v, sema).wait()

        def p1_start(ch, bufs, sem):
            base = ch * C
            pltpu.async_copy(src_hbm.at[pl.ds(base, C)], bufs[0], sem)
            pltpu.async_copy(dst_hbm.at[pl.ds(base, C)], bufs[1], sem)
            pltpu.async_copy(el_hbm.at[pl.ds(base, C)], bufs[2], sem)
            pltpu.async_copy(el2_hbm.at[pl.ds(base, C)], bufs[3], sem)

        def p1_wait(ch, bufs, sem):
            base = ch * C
            pltpu.make_async_copy(src_hbm.at[pl.ds(base, C)], bufs[0],
                                  sem).wait()
            pltpu.make_async_copy(dst_hbm.at[pl.ds(base, C)], bufs[1],
                                  sem).wait()
            pltpu.make_async_copy(el_hbm.at[pl.ds(base, C)], bufs[2],
                                  sem).wait()
            pltpu.make_async_copy(el2_hbm.at[pl.ds(base, C)], bufs[3],
                                  sem).wait()

        def p1_process(bufs, cnt):
            src_c, dst_c, el_c, el2_c = bufs

            def grp(g, cnt):
                o = g * 16
                sv = src_c[pl.ds(o, 16)]
                dv = dst_c[pl.ds(o, 16)]
                ev = el_c[pl.ds(o, 16)]
                e2v = el2_c[pl.ds(o, 16)]
                own = (dv >= lo) & (dv < lo + P)
                dof = jnp.where(own, dv - lo, 0)
                sg = plsc.load_gather(s_v, [sv])
                dg = plsc.load_gather(d_v, [dof])
                lg = sg + dg + ev
                lg = jnp.maximum(lg, lg * 0.2)
                ex = jnp.exp(lg)
                plsc.addupdate_scatter(den_v, [dof], ex, mask=own)
                plsc.store_compressed(exc_v.at[pl.ds(cnt, 16)], ex, mask=own)
                plsc.store_compressed(pkc_v.at[pl.ds(cnt, 16)],
                                      sv * 512 + dof, mask=own)
                plsc.store_compressed(el2c_v.at[pl.ds(cnt, 16)], e2v,
                                      mask=own)
                return cnt + plsc.all_reduce_population_count(own)[0]

            return plsc.parallel_loop(0, NG, unroll=4, carry=cnt)(grp)

        bufs_a = (src_va, dst_va, el_va, el2_va)
        bufs_b = (src_vb, dst_vb, el_vb, el2_vb)
        p1_start(0, bufs_a, sema)

        def p1_outer(j, cnt):
            c0 = 2 * j
            p1_wait(c0, bufs_a, sema)
            p1_start(c0 + 1, bufs_b, semb)
            cnt = p1_process(bufs_a, cnt)

            @pl.when(j < NCHUNK // 2 - 1)
            def _():
                p1_start(c0 + 2, bufs_a, sema)

            p1_wait(c0 + 1, bufs_b, semb)
            return p1_process(bufs_b, cnt)

        cnt = lax.fori_loop(0, NCHUNK // 2, p1_outer, jnp.int32(0))

        pltpu.async_copy(pkc_v, pk_hbm.at[pl.ds(wid * CAP, CAP)], semb)
        pltpu.async_copy(el2c_v, el2c_hbm.at[pl.ds(wid * CAP, CAP)], semb)
        idx_v0[pl.ds(0, 16)] = jnp.full((16,), cnt, jnp.int32)
        pltpu.async_copy(idx_v0.at[pl.ds(0, 16)],
                         cnt_hbm.at[pl.ds(wid * 16, 16)], semb)

        _alpha_pass(cnt, lanes, exc_v, pkc_v, den_v)

        pltpu.make_async_copy(pkc_v, pk_hbm.at[pl.ds(wid * CAP, CAP)],
                              semb).wait()
        pltpu.make_async_copy(el2c_v, el2c_hbm.at[pl.ds(wid * CAP, CAP)],
                              semb).wait()
        pltpu.make_async_copy(idx_v0.at[pl.ds(0, 16)],
                              cnt_hbm.at[pl.ds(wid * 16, 16)], semb).wait()

        _aggregate(cnt, lanes, lo, h4_hbm, oc_hbm, exc_v, pkc_v,
                   (idx_v0, idx_v1, idx_v2, idx_v3),
                   (rows_v0, rows_v1, rows_v2, rows_v3),
                   acc_v, (semc, semd, seme, semf))

    return k(h4, s, d, el, el2, src, dst)


def _edge_kernel2(h4, s, d, pk, el2c, cntv):
    @functools.partial(
        pl.kernel,
        mesh=_sc_mesh(),
        compiler_params=_SC_PARAMS,
        out_type=jax.ShapeDtypeStruct((NC4 * NP * 128,), jnp.float32),
        scratch_types=[
            pltpu.VMEM((NP,), jnp.float32),
            pltpu.VMEM((P,), jnp.float32),
            pltpu.VMEM((P,), jnp.float32),
            pltpu.VMEM((CAP,), jnp.float32),
            pltpu.VMEM((CAP,), jnp.int32),
            pltpu.VMEM((CAP,), jnp.float32),
            pltpu.VMEM((G,), jnp.int32),
            pltpu.VMEM((G,), jnp.int32),
            pltpu.VMEM((G,), jnp.int32),
            pltpu.VMEM((G,), jnp.int32),
            pltpu.VMEM((G, 128), jnp.float32),
            pltpu.VMEM((G, 128), jnp.float32),
            pltpu.VMEM((G, 128), jnp.float32),
            pltpu.VMEM((G, 128), jnp.float32),
            pltpu.VMEM((P * 128,), jnp.float32),
            pltpu.SemaphoreType.DMA,
            pltpu.SemaphoreType.DMA,
            pltpu.SemaphoreType.DMA,
            pltpu.SemaphoreType.DMA,
            pltpu.SemaphoreType.DMA,
            pltpu.SemaphoreType.DMA,
        ],
    )
    def k(h4_hbm, s_hbm, d_hbm, pk_hbm, el2c_hbm, cnt_hbm, oc_hbm,
          s_v, d_v, den_v, exc_v, pkc_v, el2c_v,
          idx_v0, idx_v1, idx_v2, idx_v3,
          rows_v0, rows_v1, rows_v2, rows_v3,
          acc_v, sema, semb, semc, semd, seme, semf):
        wid = lax.axis_index("s") * 2 + lax.axis_index("c")
        lo = wid * P
        pltpu.async_copy(s_hbm, s_v, sema)
        pltpu.async_copy(d_hbm.at[pl.ds(lo, P)], d_v, sema)
        pltpu.async_copy(pk_hbm.at[pl.ds(wid * CAP, CAP)], pkc_v, semb)
        pltpu.async_copy(el2c_hbm.at[pl.ds(wid * CAP, CAP)], el2c_v, semb)
        pltpu.async_copy(cnt_hbm.at[pl.ds(wid * 16, 16)],
                         idx_v0.at[pl.ds(0, 16)], semb)
        zf = jnp.zeros((16,), jnp.float32)
        lanes = lax.iota(jnp.int32, 16)

        def zden(i, carry):
            den_v[pl.ds(i * 16, 16)] = zf
            return carry

        lax.fori_loop(0, P // 16, zden, 0)
        pltpu.make_async_copy(s_hbm, s_v, sema).wait()
        pltpu.make_async_copy(d_hbm.at[pl.ds(lo, P)], d_v, sema).wait()
        pltpu.make_async_copy(pk_hbm.at[pl.ds(wid * CAP, CAP)], pkc_v,
                              semb).wait()
        pltpu.make_async_copy(el2c_hbm.at[pl.ds(wid * CAP, CAP)], el2c_v,
                              semb).wait()
        pltpu.make_async_copy(cnt_hbm.at[pl.ds(wid * 16, 16)],
                              idx_v0.at[pl.ds(0, 16)], semb).wait()
        cnt = idx_v0[pl.ds(0, 16)][0]


        def grp(g, carry):
            o = g * 16
            m = (o + lanes) < cnt
            pkv = pkc_v[pl.ds(o, 16)]
            dof = pkv & 511
            srcv = pkv >> 9
            e2v = el2c_v[pl.ds(o, 16)]
            sg = plsc.load_gather(s_v, [srcv])
            dg = plsc.load_gather(d_v, [dof])
            lg = sg + dg + e2v
            lg = jnp.maximum(lg, lg * 0.2)
            ex = jnp.where(m, jnp.exp(lg), 0.0)
            plsc.addupdate_scatter(den_v, [dof], ex, mask=m)
            exc_v[pl.ds(o, 16)] = ex
            return carry

        plsc.parallel_loop(0, CAP // 16, unroll=4)(
            lambda g: grp(g, 0) and None)

        _alpha_pass(cnt, lanes, exc_v, pkc_v, den_v)
        _aggregate(cnt, lanes, lo, h4_hbm, oc_hbm, exc_v, pkc_v,
                   (idx_v0, idx_v1, idx_v2, idx_v3),
                   (rows_v0, rows_v1, rows_v2, rows_v3),
                   acc_v, (semc, semd, seme, semf))

    return k(h4, s, d, pk, el2c, cntv)


def kernel(x, edge_index, edge_attr, W1, a_src1, a_dst1, We1, ae1, b1,
           W2, a_src2, a_dst2, We2, ae2, b2):
    x_pad = jnp.zeros((NP, DIN), jnp.float32).at[:N].set(x)
    src = edge_index[0]
    dst = edge_index[1]
    ea_pad = jnp.zeros((EP, DE), jnp.float32).at[:E].set(edge_attr)
    el1, el2 = _edge_logits(ea_pad.reshape(ER, 64 * DE), We1, ae1, We2, ae2)

    h1, s1, d1 = _projection(x_pad, W1, a_src1, a_dst1,
                             jnp.zeros((H,), jnp.float32), False)
    oc1, pk, el2c, cntv = _edge_kernel1(h1.reshape(NP * NC4, 128), s1, d1,
                                        el1, el2, src, dst)
    o1 = oc1.reshape(NC4, NP, 128).transpose(1, 0, 2).reshape(NP, H)

    h2, s2, d2 = _projection(o1, W2, a_src2, a_dst2, b1, True)
    oc2 = _edge_kernel2(h2.reshape(NP * NC4, 128), s2, d2, pk, el2c, cntv)
    out = oc2.reshape(NC4, NP, 128).transpose(1, 0, 2).reshape(NP, H)[:N] + b2
    return out
